# baseline (speedup 1.0000x reference)
import jax
import jax.numpy as jnp
from jax import lax
from jax.experimental import pallas as pl
from jax.experimental.pallas import tpu as pltpu
from jax.experimental.pallas import tpu_sc as plsc

N_NODES = 10000
N_EDGES = 320000
D = 128

NC = 1
NS = 16
NW = NC * NS
K = 128
STEPS = 160
E_PAD = NW * K * STEPS
N_ACC = 10112
RPT = N_ACC // NS

ROW_BLK = 1000
DEG_BLK = 3200
DEG_Q = 80


def _sc_agg_body(table, srcs, dsts, zrows, out_acc,
                 acc_sh, sb, db, rows0, rows1,
                 ssem0, ssem1, ssem2, ssem3,
                 dsem0, dsem1, dsem2, dsem3, sem0, sem1):
    c = lax.axis_index("c")
    s = lax.axis_index("s")
    wid = c * NS + s

    ssem = (ssem0, ssem1, ssem2, ssem3)
    dsem = (dsem0, dsem1, dsem2, dsem3)
    rows = (rows0, rows1)
    gsem = (sem0, sem1)

    def pf(j, slot):
        pltpu.async_copy(srcs.at[wid, j], sb.at[slot], ssem[slot])
        pltpu.async_copy(dsts.at[wid, j], db.at[slot], dsem[slot])

    def gather(slot):
        pltpu.make_async_copy(srcs.at[wid, 0], sb.at[slot], ssem[slot]).wait()
        pltpu.async_copy(table.at[sb.at[slot]], rows[slot % 2], gsem[slot % 2])

    def scat(slot):
        pltpu.make_async_copy(table.at[sb.at[slot]], rows[slot % 2],
                              gsem[slot % 2]).wait()
        pltpu.make_async_copy(dsts.at[wid, 0], db.at[slot], dsem[slot]).wait()
        pltpu.sync_copy(rows[slot % 2], acc_sh.at[db.at[slot]], add=True)

    pltpu.sync_copy(zrows.at[pl.ds(s * RPT, RPT)],
                    acc_sh.at[pl.ds(s * RPT, RPT)])
    plsc.subcore_barrier()

    for slot in range(4):
        pf(slot, slot)
    gather(0)
    gather(1)

    G4 = STEPS // 4

    def step4(g, carry):
        j0 = 4 * g
        last = g + 1 >= G4

        scat(0)
        gather(2)

        @pl.when(~last)
        def _():
            pf(j0 + 4, 0)

        scat(1)
        gather(3)

        @pl.when(~last)
        def _():
            pf(j0 + 5, 1)

        scat(2)

        @pl.when(~last)
        def _():
            gather(0)
            pf(j0 + 6, 2)

        scat(3)

        @pl.when(~last)
        def _():
            gather(1)
            pf(j0 + 7, 3)

        return carry

    lax.fori_loop(0, G4, step4, 0)
    plsc.subcore_barrier()

    pltpu.sync_copy(acc_sh.at[pl.ds(s * RPT, RPT)],
                    out_acc.at[c, pl.ds(s * RPT, RPT)])


def _make_sc_agg():
    mesh = plsc.VectorSubcoreMesh(core_axis_name="c", subcore_axis_name="s",
                                  num_cores=NC)
    return pl.kernel(
        _sc_agg_body,
        out_type=jax.ShapeDtypeStruct((NC, N_ACC, D), jnp.float32),
        mesh=mesh,
        scratch_types=[
            pltpu.VMEM_SHARED((N_ACC, D), jnp.float32),
            pltpu.VMEM((4, K), jnp.int32),
            pltpu.VMEM((4, K), jnp.int32),
            pltpu.VMEM((K, D), jnp.float32),
            pltpu.VMEM((K, D), jnp.float32),
        ] + [pltpu.SemaphoreType.DMA] * 10,
    )


def _deg_body(dst_ref, inv_ref):
    i = pl.program_id(0)
    d = dst_ref[0, 0, :]
    q = jax.lax.shift_right_logical(d, 7)
    r = jax.lax.bitwise_and(d, 127)
    qcols = jax.lax.broadcasted_iota(jnp.int32, (DEG_BLK, DEG_Q), 1)
    rcols = jax.lax.broadcasted_iota(jnp.int32, (DEG_BLK, D), 1)
    conehot = (q[:, None] == qcols).astype(jnp.float32)
    ronehot = (r[:, None] == rcols).astype(jnp.float32)
    partial = jax.lax.dot_general(
        conehot, ronehot, (((0,), (0,)), ((), ())),
        preferred_element_type=jnp.float32)

    @pl.when(i == 0)
    def _():
        inv_ref[...] = jnp.zeros_like(inv_ref)

    inv_ref[...] += partial

    @pl.when(i == pl.num_programs(0) - 1)
    def _():
        inv_ref[...] = 1.0 / jnp.maximum(inv_ref[...], 1.0)


def _tc_inv_deg(dst):
    dst3 = dst.reshape(N_EDGES // DEG_BLK, 1, DEG_BLK)
    return pl.pallas_call(
        _deg_body,
        grid=(N_EDGES // DEG_BLK,),
        in_specs=[pl.BlockSpec((1, 1, DEG_BLK), lambda i: (i, 0, 0))],
        out_specs=pl.BlockSpec((DEG_Q, D), lambda i: (0, 0)),
        out_shape=jax.ShapeDtypeStruct((DEG_Q, D), jnp.float32),
    )(dst3)


def _mm2_body(x_ref, wl_ref, wr_ref, y_ref, z_ref):
    x = x_ref[...]
    y_ref[...] = jnp.dot(x, wl_ref[...], preferred_element_type=jnp.float32)
    z_ref[...] = jnp.dot(x, wr_ref[...], preferred_element_type=jnp.float32)


def _tc_dual_matmul(x, wl, wr):
    return pl.pallas_call(
        _mm2_body,
        grid=(N_NODES // ROW_BLK,),
        in_specs=[
            pl.BlockSpec((ROW_BLK, D), lambda i: (i, 0)),
            pl.BlockSpec((D, D), lambda i: (0, 0)),
            pl.BlockSpec((D, D), lambda i: (0, 0)),
        ],
        out_specs=[
            pl.BlockSpec((ROW_BLK, D), lambda i: (i, 0)),
            pl.BlockSpec((ROW_BLK, D), lambda i: (i, 0)),
        ],
        out_shape=[
            jax.ShapeDtypeStruct((N_NODES, D), jnp.float32),
            jax.ShapeDtypeStruct((N_NODES, D), jnp.float32),
        ],
    )(x, wl, wr)


def _combine1_body(acc_ref, inv_ref, z_ref, b_ref, wl_ref, wr_ref,
                   y2_ref, z2_ref):
    agg = jnp.sum(acc_ref[...], axis=0)
    inv = inv_ref[...]
    h = jnp.maximum(agg * inv + z_ref[...] + b_ref[...], 0.0)
    y2_ref[...] = jnp.dot(h, wl_ref[...], preferred_element_type=jnp.float32)
    z2_ref[...] = jnp.dot(h, wr_ref[...], preferred_element_type=jnp.float32)


def _tc_combine1(acc, inv, z1, b1, wl2, wr2):
    return pl.pallas_call(
        _combine1_body,
        grid=(N_NODES // ROW_BLK,),
        in_specs=[
            pl.BlockSpec((NC, ROW_BLK, D), lambda i: (0, i, 0)),
            pl.BlockSpec((ROW_BLK, 1), lambda i: (i, 0)),
            pl.BlockSpec((ROW_BLK, D), lambda i: (i, 0)),
            pl.BlockSpec((1, D), lambda i: (0, 0)),
            pl.BlockSpec((D, D), lambda i: (0, 0)),
            pl.BlockSpec((D, D), lambda i: (0, 0)),
        ],
        out_specs=[
            pl.BlockSpec((ROW_BLK, D), lambda i: (i, 0)),
            pl.BlockSpec((ROW_BLK, D), lambda i: (i, 0)),
        ],
        out_shape=[
            jax.ShapeDtypeStruct((N_NODES, D), jnp.float32),
            jax.ShapeDtypeStruct((N_NODES, D), jnp.float32),
        ],
    )(acc, inv, z1, b1, wl2, wr2)


def _combine2_body(acc_ref, inv_ref, z_ref, b_ref, out_ref):
    agg = jnp.sum(acc_ref[...], axis=0)
    out_ref[...] = agg * inv_ref[...] + z_ref[...] + b_ref[...]


def _tc_combine2(acc, inv, z2, b2):
    return pl.pallas_call(
        _combine2_body,
        grid=(N_NODES // ROW_BLK,),
        in_specs=[
            pl.BlockSpec((NC, ROW_BLK, D), lambda i: (0, i, 0)),
            pl.BlockSpec((ROW_BLK, 1), lambda i: (i, 0)),
            pl.BlockSpec((ROW_BLK, D), lambda i: (i, 0)),
            pl.BlockSpec((1, D), lambda i: (0, 0)),
        ],
        out_specs=pl.BlockSpec((ROW_BLK, D), lambda i: (i, 0)),
        out_shape=jax.ShapeDtypeStruct((N_NODES, D), jnp.float32),
    )(acc, inv, z2, b2)


def kernel(x, edge_index, W_l1, W_r1, b1, W_l2, W_r2, b2):
    ei = edge_index.astype(jnp.int32)
    pad = E_PAD - N_EDGES
    src_p = jnp.concatenate(
        [ei[0], jnp.zeros((pad,), jnp.int32)]).reshape(NW, STEPS, K)
    dst_p = jnp.concatenate(
        [ei[1], jnp.full((pad,), N_NODES, jnp.int32)]).reshape(NW, STEPS, K)
    zrows = jnp.zeros((N_ACC, D), jnp.float32)

    inv = _tc_inv_deg(ei[1]).reshape(-1)[:N_NODES].reshape(N_NODES, 1)

    sc_agg = _make_sc_agg()
    y1, z1 = _tc_dual_matmul(x, W_l1, W_r1)
    acc1 = sc_agg(y1, src_p, dst_p, zrows)
    y2, z2 = _tc_combine1(acc1, inv, z1, b1.reshape(1, D), W_l2, W_r2)
    acc2 = sc_agg(y2, src_p, dst_p, zrows)
    return _tc_combine2(acc2, inv, z2, b2.reshape(1, D))


# speedup vs baseline: 1.0323x; 1.0323x over previous
import jax
import jax.numpy as jnp
from jax import lax
from jax.experimental import pallas as pl
from jax.experimental.pallas import tpu as pltpu
from jax.experimental.pallas import tpu_sc as plsc

N_NODES = 10000
N_EDGES = 320000
D = 128

NC = 1
NS = 16
NW = NC * NS
K = 128
STEPS = 160
E_PAD = NW * K * STEPS
N_ACC = 10112
RPT = N_ACC // NS

ROW_BLK = 1000
DEG_BLK = 3200
DEG_Q = 80


HALF = STEPS // 2


def _sc_agg_body(table, srcs, dsts, zrows, out_acc,
                 acc_sh, src_a, dst_b, rows0, rows1,
                 isem, sem0, sem1, dsem0, dsem1):
    c = lax.axis_index("c")
    s = lax.axis_index("s")
    wid = c * NS + s

    pltpu.sync_copy(zrows.at[pl.ds(s * RPT, RPT)],
                    acc_sh.at[pl.ds(s * RPT, RPT)])
    plsc.subcore_barrier()

    for h in range(2):
        pltpu.async_copy(srcs.at[wid, h], src_a, isem)
        pltpu.make_async_copy(srcs.at[wid, h], src_a, isem).wait()
        off = h * HALF

        pltpu.async_copy(table.at[src_a.at[0]], rows0, sem0)
        pltpu.async_copy(dsts.at[wid, off], dst_b.at[0], dsem0)

        def step2(g, carry):
            i0 = g * 2
            pltpu.async_copy(table.at[src_a.at[i0 + 1]], rows1, sem1)
            pltpu.async_copy(dsts.at[wid, off + i0 + 1], dst_b.at[1], dsem1)
            pltpu.make_async_copy(table.at[src_a.at[i0]], rows0, sem0).wait()
            pltpu.make_async_copy(dsts.at[wid, off], dst_b.at[0], dsem0).wait()
            pltpu.sync_copy(rows0, acc_sh.at[dst_b.at[0]], add=True)

            @pl.when(g + 1 < HALF // 2)
            def _():
                pltpu.async_copy(table.at[src_a.at[i0 + 2]], rows0, sem0)
                pltpu.async_copy(dsts.at[wid, off + i0 + 2], dst_b.at[0],
                                 dsem0)

            pltpu.make_async_copy(table.at[src_a.at[i0 + 1]], rows1,
                                  sem1).wait()
            pltpu.make_async_copy(dsts.at[wid, off], dst_b.at[1],
                                  dsem1).wait()
            pltpu.sync_copy(rows1, acc_sh.at[dst_b.at[1]], add=True)
            return carry

        lax.fori_loop(0, HALF // 2, step2, 0)

    plsc.subcore_barrier()

    pltpu.sync_copy(acc_sh.at[pl.ds(s * RPT, RPT)],
                    out_acc.at[c, pl.ds(s * RPT, RPT)])


def _make_sc_agg():
    mesh = plsc.VectorSubcoreMesh(core_axis_name="c", subcore_axis_name="s",
                                  num_cores=NC)
    return pl.kernel(
        _sc_agg_body,
        out_type=jax.ShapeDtypeStruct((NC, N_ACC, D), jnp.float32),
        mesh=mesh,
        scratch_types=[
            pltpu.VMEM_SHARED((N_ACC, D), jnp.float32),
            pltpu.VMEM((HALF, K), jnp.int32),
            pltpu.VMEM((2, K), jnp.int32),
            pltpu.VMEM((K, D), jnp.float32),
            pltpu.VMEM((K, D), jnp.float32),
        ] + [pltpu.SemaphoreType.DMA] * 5,
    )


def _deg_body(dst_ref, inv_ref):
    i = pl.program_id(0)
    d = dst_ref[0, 0, :]
    q = jax.lax.shift_right_logical(d, 7)
    r = jax.lax.bitwise_and(d, 127)
    qcols = jax.lax.broadcasted_iota(jnp.int32, (DEG_BLK, DEG_Q), 1)
    rcols = jax.lax.broadcasted_iota(jnp.int32, (DEG_BLK, D), 1)
    conehot = (q[:, None] == qcols).astype(jnp.float32)
    ronehot = (r[:, None] == rcols).astype(jnp.float32)
    partial = jax.lax.dot_general(
        conehot, ronehot, (((0,), (0,)), ((), ())),
        preferred_element_type=jnp.float32)

    @pl.when(i == 0)
    def _():
        inv_ref[...] = jnp.zeros_like(inv_ref)

    inv_ref[...] += partial

    @pl.when(i == pl.num_programs(0) - 1)
    def _():
        inv_ref[...] = 1.0 / jnp.maximum(inv_ref[...], 1.0)


def _tc_inv_deg(dst):
    dst3 = dst.reshape(N_EDGES // DEG_BLK, 1, DEG_BLK)
    return pl.pallas_call(
        _deg_body,
        grid=(N_EDGES // DEG_BLK,),
        in_specs=[pl.BlockSpec((1, 1, DEG_BLK), lambda i: (i, 0, 0))],
        out_specs=pl.BlockSpec((DEG_Q, D), lambda i: (0, 0)),
        out_shape=jax.ShapeDtypeStruct((DEG_Q, D), jnp.float32),
    )(dst3)


def _mm2_body(x_ref, wl_ref, wr_ref, y_ref, z_ref):
    x = x_ref[...]
    y_ref[...] = jnp.dot(x, wl_ref[...], preferred_element_type=jnp.float32)
    z_ref[...] = jnp.dot(x, wr_ref[...], preferred_element_type=jnp.float32)


def _tc_dual_matmul(x, wl, wr):
    return pl.pallas_call(
        _mm2_body,
        grid=(N_NODES // ROW_BLK,),
        in_specs=[
            pl.BlockSpec((ROW_BLK, D), lambda i: (i, 0)),
            pl.BlockSpec((D, D), lambda i: (0, 0)),
            pl.BlockSpec((D, D), lambda i: (0, 0)),
        ],
        out_specs=[
            pl.BlockSpec((ROW_BLK, D), lambda i: (i, 0)),
            pl.BlockSpec((ROW_BLK, D), lambda i: (i, 0)),
        ],
        out_shape=[
            jax.ShapeDtypeStruct((N_NODES, D), jnp.float32),
            jax.ShapeDtypeStruct((N_NODES, D), jnp.float32),
        ],
    )(x, wl, wr)


def _combine1_body(acc_ref, inv_ref, z_ref, b_ref, wl_ref, wr_ref,
                   y2_ref, z2_ref):
    agg = jnp.sum(acc_ref[...], axis=0)
    inv = inv_ref[...]
    h = jnp.maximum(agg * inv + z_ref[...] + b_ref[...], 0.0)
    y2_ref[...] = jnp.dot(h, wl_ref[...], preferred_element_type=jnp.float32)
    z2_ref[...] = jnp.dot(h, wr_ref[...], preferred_element_type=jnp.float32)


def _tc_combine1(acc, inv, z1, b1, wl2, wr2):
    return pl.pallas_call(
        _combine1_body,
        grid=(N_NODES // ROW_BLK,),
        in_specs=[
            pl.BlockSpec((NC, ROW_BLK, D), lambda i: (0, i, 0)),
            pl.BlockSpec((ROW_BLK, 1), lambda i: (i, 0)),
            pl.BlockSpec((ROW_BLK, D), lambda i: (i, 0)),
            pl.BlockSpec((1, D), lambda i: (0, 0)),
            pl.BlockSpec((D, D), lambda i: (0, 0)),
            pl.BlockSpec((D, D), lambda i: (0, 0)),
        ],
        out_specs=[
            pl.BlockSpec((ROW_BLK, D), lambda i: (i, 0)),
            pl.BlockSpec((ROW_BLK, D), lambda i: (i, 0)),
        ],
        out_shape=[
            jax.ShapeDtypeStruct((N_NODES, D), jnp.float32),
            jax.ShapeDtypeStruct((N_NODES, D), jnp.float32),
        ],
    )(acc, inv, z1, b1, wl2, wr2)


def _combine2_body(acc_ref, inv_ref, z_ref, b_ref, out_ref):
    agg = jnp.sum(acc_ref[...], axis=0)
    out_ref[...] = agg * inv_ref[...] + z_ref[...] + b_ref[...]


def _tc_combine2(acc, inv, z2, b2):
    return pl.pallas_call(
        _combine2_body,
        grid=(N_NODES // ROW_BLK,),
        in_specs=[
            pl.BlockSpec((NC, ROW_BLK, D), lambda i: (0, i, 0)),
            pl.BlockSpec((ROW_BLK, 1), lambda i: (i, 0)),
            pl.BlockSpec((ROW_BLK, D), lambda i: (i, 0)),
            pl.BlockSpec((1, D), lambda i: (0, 0)),
        ],
        out_specs=pl.BlockSpec((ROW_BLK, D), lambda i: (i, 0)),
        out_shape=jax.ShapeDtypeStruct((N_NODES, D), jnp.float32),
    )(acc, inv, z2, b2)


def kernel(x, edge_index, W_l1, W_r1, b1, W_l2, W_r2, b2):
    ei = edge_index.astype(jnp.int32)
    pad = E_PAD - N_EDGES
    src_p = jnp.concatenate(
        [ei[0], jnp.zeros((pad,), jnp.int32)]).reshape(NW, 2, HALF, K)
    dst_p = jnp.concatenate(
        [ei[1], jnp.full((pad,), N_NODES, jnp.int32)]).reshape(NW, STEPS, K)
    zrows = jnp.zeros((N_ACC, D), jnp.float32)

    inv = _tc_inv_deg(ei[1]).reshape(-1)[:N_NODES].reshape(N_NODES, 1)

    sc_agg = _make_sc_agg()
    y1, z1 = _tc_dual_matmul(x, W_l1, W_r1)
    acc1 = sc_agg(y1, src_p, dst_p, zrows)
    y2, z2 = _tc_combine1(acc1, inv, z1, b1.reshape(1, D), W_l2, W_r2)
    acc2 = sc_agg(y2, src_p, dst_p, zrows)
    return _tc_combine2(acc2, inv, z2, b2.reshape(1, D))


# speedup vs baseline: 1.2453x; 1.2063x over previous
import jax
import jax.numpy as jnp
from jax import lax
from jax.experimental import pallas as pl
from jax.experimental.pallas import tpu as pltpu
from jax.experimental.pallas import tpu_sc as plsc

N_NODES = 10000
N_EDGES = 320000
D = 128

NC = 2
NS = 16
NW = NC * NS
K = 128
CHUNK = 32
S0 = 128
S1 = 32
NCHUNK0 = S0 // CHUNK
NCHUNK1 = S1 // CHUNK
ROWS = NS * (S0 + S1)
E_PAD = ROWS * K
N_ACC = 10112
RPT = N_ACC // NS

ROW_BLK = 1000
DEG_BLK = 3200
DEG_Q = 80


def _sc_agg_body(table, srcs, dsts, zrows, out_acc,
                 acc_sh, src_a, dst_b, rows0, rows1,
                 isem, sem0, sem1, dsem0, dsem1):
    c = lax.axis_index("c")
    s = lax.axis_index("s")

    base = jnp.where(c == 0, s * S0, NS * S0 + s * S1)

    pltpu.sync_copy(zrows.at[pl.ds(s * RPT, RPT)],
                    acc_sh.at[pl.ds(s * RPT, RPT)])
    plsc.subcore_barrier()

    nchunk = jnp.where(c == 0, NCHUNK0, NCHUNK1)

    for ch in range(NCHUNK0):
        @pl.when(ch < nchunk)
        def _():
            off = base + ch * CHUNK
            pltpu.async_copy(srcs.at[pl.ds(off, CHUNK)], src_a, isem)
            pltpu.make_async_copy(srcs.at[pl.ds(off, CHUNK)], src_a,
                                  isem).wait()

            pltpu.async_copy(table.at[src_a.at[0]], rows0, sem0)
            pltpu.async_copy(dsts.at[off], dst_b.at[0], dsem0)

            def step2(g, carry):
                i0 = g * 2
                pltpu.async_copy(table.at[src_a.at[i0 + 1]], rows1, sem1)
                pltpu.async_copy(dsts.at[off + i0 + 1], dst_b.at[1], dsem1)
                pltpu.make_async_copy(table.at[src_a.at[i0]], rows0,
                                      sem0).wait()
                pltpu.make_async_copy(dsts.at[off], dst_b.at[0],
                                      dsem0).wait()
                pltpu.sync_copy(rows0, acc_sh.at[dst_b.at[0]], add=True)

                @pl.when(g + 1 < CHUNK // 2)
                def _():
                    pltpu.async_copy(table.at[src_a.at[i0 + 2]], rows0, sem0)
                    pltpu.async_copy(dsts.at[off + i0 + 2], dst_b.at[0],
                                     dsem0)

                pltpu.make_async_copy(table.at[src_a.at[i0 + 1]], rows1,
                                      sem1).wait()
                pltpu.make_async_copy(dsts.at[off], dst_b.at[1],
                                      dsem1).wait()
                pltpu.sync_copy(rows1, acc_sh.at[dst_b.at[1]], add=True)
                return carry

            lax.fori_loop(0, CHUNK // 2, step2, 0)

    plsc.subcore_barrier()

    pltpu.sync_copy(acc_sh.at[pl.ds(s * RPT, RPT)],
                    out_acc.at[c, pl.ds(s * RPT, RPT)])


def _make_sc_agg():
    mesh = plsc.VectorSubcoreMesh(core_axis_name="c", subcore_axis_name="s",
                                  num_cores=NC)
    return pl.kernel(
        _sc_agg_body,
        out_type=jax.ShapeDtypeStruct((NC, N_ACC, D), jnp.float32),
        mesh=mesh,
        scratch_types=[
            pltpu.VMEM_SHARED((N_ACC, D), jnp.float32),
            pltpu.VMEM((CHUNK, K), jnp.int32),
            pltpu.VMEM((2, K), jnp.int32),
            pltpu.VMEM((K, D), jnp.float32),
            pltpu.VMEM((K, D), jnp.float32),
        ] + [pltpu.SemaphoreType.DMA] * 5,
    )


def _deg_body(dst_ref, inv_ref):
    i = pl.program_id(0)
    d = dst_ref[0, 0, :]
    q = jax.lax.shift_right_logical(d, 7)
    r = jax.lax.bitwise_and(d, 127)
    qcols = jax.lax.broadcasted_iota(jnp.int32, (DEG_BLK, DEG_Q), 1)
    rcols = jax.lax.broadcasted_iota(jnp.int32, (DEG_BLK, D), 1)
    conehot = (q[:, None] == qcols).astype(jnp.float32)
    ronehot = (r[:, None] == rcols).astype(jnp.float32)
    partial = jax.lax.dot_general(
        conehot, ronehot, (((0,), (0,)), ((), ())),
        preferred_element_type=jnp.float32)

    @pl.when(i == 0)
    def _():
        inv_ref[...] = jnp.zeros_like(inv_ref)

    inv_ref[...] += partial

    @pl.when(i == pl.num_programs(0) - 1)
    def _():
        inv_ref[...] = 1.0 / jnp.maximum(inv_ref[...], 1.0)


def _tc_inv_deg(dst):
    dst3 = dst.reshape(N_EDGES // DEG_BLK, 1, DEG_BLK)
    return pl.pallas_call(
        _deg_body,
        grid=(N_EDGES // DEG_BLK,),
        in_specs=[pl.BlockSpec((1, 1, DEG_BLK), lambda i: (i, 0, 0))],
        out_specs=pl.BlockSpec((DEG_Q, D), lambda i: (0, 0)),
        out_shape=jax.ShapeDtypeStruct((DEG_Q, D), jnp.float32),
    )(dst3)


def _mm2_body(x_ref, wl_ref, wr_ref, y_ref, z_ref):
    x = x_ref[...]
    y_ref[...] = jnp.dot(x, wl_ref[...], preferred_element_type=jnp.float32)
    z_ref[...] = jnp.dot(x, wr_ref[...], preferred_element_type=jnp.float32)


def _tc_dual_matmul(x, wl, wr):
    return pl.pallas_call(
        _mm2_body,
        grid=(N_NODES // ROW_BLK,),
        in_specs=[
            pl.BlockSpec((ROW_BLK, D), lambda i: (i, 0)),
            pl.BlockSpec((D, D), lambda i: (0, 0)),
            pl.BlockSpec((D, D), lambda i: (0, 0)),
        ],
        out_specs=[
            pl.BlockSpec((ROW_BLK, D), lambda i: (i, 0)),
            pl.BlockSpec((ROW_BLK, D), lambda i: (i, 0)),
        ],
        out_shape=[
            jax.ShapeDtypeStruct((N_NODES, D), jnp.float32),
            jax.ShapeDtypeStruct((N_NODES, D), jnp.float32),
        ],
    )(x, wl, wr)


def _combine1_body(acc_ref, inv_ref, z_ref, b_ref, wl_ref, wr_ref,
                   y2_ref, z2_ref):
    agg = jnp.sum(acc_ref[...], axis=0)
    inv = inv_ref[...]
    h = jnp.maximum(agg * inv + z_ref[...] + b_ref[...], 0.0)
    y2_ref[...] = jnp.dot(h, wl_ref[...], preferred_element_type=jnp.float32)
    z2_ref[...] = jnp.dot(h, wr_ref[...], preferred_element_type=jnp.float32)


def _tc_combine1(acc, inv, z1, b1, wl2, wr2):
    return pl.pallas_call(
        _combine1_body,
        grid=(N_NODES // ROW_BLK,),
        in_specs=[
            pl.BlockSpec((NC, ROW_BLK, D), lambda i: (0, i, 0)),
            pl.BlockSpec((ROW_BLK, 1), lambda i: (i, 0)),
            pl.BlockSpec((ROW_BLK, D), lambda i: (i, 0)),
            pl.BlockSpec((1, D), lambda i: (0, 0)),
            pl.BlockSpec((D, D), lambda i: (0, 0)),
            pl.BlockSpec((D, D), lambda i: (0, 0)),
        ],
        out_specs=[
            pl.BlockSpec((ROW_BLK, D), lambda i: (i, 0)),
            pl.BlockSpec((ROW_BLK, D), lambda i: (i, 0)),
        ],
        out_shape=[
            jax.ShapeDtypeStruct((N_NODES, D), jnp.float32),
            jax.ShapeDtypeStruct((N_NODES, D), jnp.float32),
        ],
    )(acc, inv, z1, b1, wl2, wr2)


def _combine2_body(acc_ref, inv_ref, z_ref, b_ref, out_ref):
    agg = jnp.sum(acc_ref[...], axis=0)
    out_ref[...] = agg * inv_ref[...] + z_ref[...] + b_ref[...]


def _tc_combine2(acc, inv, z2, b2):
    return pl.pallas_call(
        _combine2_body,
        grid=(N_NODES // ROW_BLK,),
        in_specs=[
            pl.BlockSpec((NC, ROW_BLK, D), lambda i: (0, i, 0)),
            pl.BlockSpec((ROW_BLK, 1), lambda i: (i, 0)),
            pl.BlockSpec((ROW_BLK, D), lambda i: (i, 0)),
            pl.BlockSpec((1, D), lambda i: (0, 0)),
        ],
        out_specs=pl.BlockSpec((ROW_BLK, D), lambda i: (i, 0)),
        out_shape=jax.ShapeDtypeStruct((N_NODES, D), jnp.float32),
    )(acc, inv, z2, b2)


def kernel(x, edge_index, W_l1, W_r1, b1, W_l2, W_r2, b2):
    ei = edge_index.astype(jnp.int32)
    pad = E_PAD - N_EDGES
    src_p = jnp.concatenate(
        [ei[0], jnp.zeros((pad,), jnp.int32)]).reshape(ROWS, K)
    dst_p = jnp.concatenate(
        [ei[1], jnp.full((pad,), N_NODES, jnp.int32)]).reshape(ROWS, K)
    zrows = jnp.zeros((N_ACC, D), jnp.float32)

    inv = _tc_inv_deg(ei[1]).reshape(-1)[:N_NODES].reshape(N_NODES, 1)

    sc_agg = _make_sc_agg()
    y1, z1 = _tc_dual_matmul(x, W_l1, W_r1)
    acc1 = sc_agg(y1, src_p, dst_p, zrows)
    y2, z2 = _tc_combine1(acc1, inv, z1, b1.reshape(1, D), W_l2, W_r2)
    acc2 = sc_agg(y2, src_p, dst_p, zrows)
    return _tc_combine2(acc2, inv, z2, b2.reshape(1, D))
